# Initial kernel scaffold; baseline (speedup 1.0000x reference)
import jax
import jax.numpy as jnp
from jax.experimental import pallas as pl


def kernel(img, kernel_sizes, readout_noise_base, rgb_QEs, vignet_gain, aggregator_gain, dark_currents, noise_gains, STD_reads, expsr2dv_gains, expsr2dv_biases, expsr2dv_gamma):
    raise NotImplementedError("write your pallas kernel here")


# speedup vs baseline: 328.6357x; 328.6357x over previous
import functools
import math

import jax
import jax.numpy as jnp
import numpy as np
from jax.experimental import pallas as pl
from jax.experimental.pallas import tpu as pltpu

H = 1080
W = 1920
C = 3
WL = W * C
MAX_HALF_K = 3
BR = 72
NBLK8 = H // 8

_G = np.zeros((8, 4), np.float32)
for _k in range(8):
    if _k <= 1:
        _G[_k, 0] = 1.0
    else:
        _s = _k / 6.0
        _hk = _k // 2
        for _d in range(4):
            if _d <= _hk:
                _G[_k, _d] = math.exp(-_d * _d / (2.0 * _s * _s)) / (
                    math.sqrt(2.0 * math.pi) * _s)


def _gsel(ksf, j):
    w = jnp.full(ksf.shape, _G[7, j], dtype=jnp.float32)
    for k in range(6, -1, -1):
        w = jnp.where(ksf < (k + 0.5), _G[k, j], w)
    return w


def _camera_kernel(params, img_c, img_t, img_b, ks_c, ks_t, ks_b, nb, out_ref):
    i = pl.program_id(0)
    n = pl.num_programs(0)

    qe_row = params[0:1, :]
    dark_row = params[1:2, :]
    ngain_row = params[2:3, :]
    std_row = params[3:4, :]
    egain_row = params[4:5, :]
    ebias_row = params[5:6, :]
    xx2_row = params[6:7, :]
    vg_row = params[7:8, :]
    agg_row = params[8:9, :]
    invg_row = params[9:10, :]

    top_e = jnp.where(i > 0, img_t[5:8, :], 0.0)
    bot_e = jnp.where(i < n - 1, img_b[0:3, :], 0.0)
    eh = jnp.concatenate([top_e, img_c[:, :], bot_e], axis=0)
    ksh = jnp.concatenate([ks_t[5:8, :], ks_c[:, :], ks_b[0:3, :]], axis=0)

    rows = jax.lax.broadcasted_iota(jnp.int32, (BR + 6, WL), 0)
    gy = rows.astype(jnp.float32) + (i * BR - 3).astype(jnp.float32)
    yy = (gy - (H - 1) / 2.0) / (W / 2.0)
    den = 1.0 + vg_row * (yy * yy + xx2_row)
    e = eh * qe_row / (den * den)

    g = [_gsel(ksh, j) for j in range(4)]
    u = [e * g[j] for j in range(4)]
    m = {}
    for j in range(4):
        for k in range(j, 4):
            m[(j, k)] = u[j] * g[k]

    def mm(a, b):
        return m[(a, b) if a <= b else (b, a)]

    v = []
    for k2 in range(4):
        acc = mm(0, k2)[3:3 + BR, :]
        for d in range(1, 4):
            acc = acc + mm(d, k2)[3 - d:3 - d + BR, :]
            acc = acc + mm(d, k2)[3 + d:3 + d + BR, :]
        v.append(acc)

    lane = jax.lax.broadcasted_iota(jnp.int32, (BR, WL), 1)
    out = v[0]
    for d in range(1, 4):
        s = 3 * d
        out = out + jnp.where(lane >= s, pltpu.roll(v[d], s, 1), 0.0)
        out = out + jnp.where(lane < WL - s, pltpu.roll(v[d], WL - s, 1), 0.0)

    nbv = nb[:, :]
    eb = out * agg_row
    shot = ngain_row * jnp.sqrt(jnp.maximum(eb, 1e-6)) * nbv
    val = eb + dark_row + shot + std_row * nbv
    lin = jnp.maximum(egain_row * val + ebias_row, 0.0)
    dv = jnp.exp(jnp.log(lin + 1e-6) * invg_row)
    out_ref[:, :] = jnp.clip(dv, 0.0, 1.0)


@jax.jit
def _run(img, kernel_sizes, readout_noise_base, rgb_QEs, vignet_gain,
         aggregator_gain, dark_currents, noise_gains, STD_reads,
         expsr2dv_gains, expsr2dv_biases, expsr2dv_gamma):
    img_r = img.reshape(H, WL)
    nb_r = readout_noise_base.reshape(H, WL)
    ks3 = jnp.repeat(kernel_sizes.astype(jnp.float32), C, axis=1)

    xx = (np.arange(W, dtype=np.float32) - (W - 1) / 2.0) / (W / 2.0)
    xx2 = jnp.asarray(np.repeat(xx * xx, C))
    def tile_c(p):
        return jnp.tile(p.astype(jnp.float32), W)
    params = jnp.stack([
        tile_c(rgb_QEs),
        tile_c(dark_currents),
        tile_c(noise_gains),
        tile_c(STD_reads),
        tile_c(expsr2dv_gains),
        tile_c(expsr2dv_biases),
        xx2,
        jnp.full((WL,), vignet_gain, jnp.float32),
        jnp.full((WL,), aggregator_gain, jnp.float32),
        jnp.full((WL,), 1.0 / expsr2dv_gamma, jnp.float32),
        jnp.zeros((WL,), jnp.float32),
        jnp.zeros((WL,), jnp.float32),
        jnp.zeros((WL,), jnp.float32),
        jnp.zeros((WL,), jnp.float32),
        jnp.zeros((WL,), jnp.float32),
        jnp.zeros((WL,), jnp.float32),
    ])

    n = H // BR
    mblk = BR // 8
    main_spec = pl.BlockSpec((BR, WL), lambda i: (i, 0))
    top_spec = pl.BlockSpec((8, WL), lambda i: (jnp.maximum(i * mblk - 1, 0), 0))
    bot_spec = pl.BlockSpec(
        (8, WL), lambda i: (jnp.minimum((i + 1) * mblk, NBLK8 - 1), 0))
    param_spec = pl.BlockSpec((16, WL), lambda i: (0, 0))

    out = pl.pallas_call(
        _camera_kernel,
        grid=(n,),
        in_specs=[param_spec,
                  main_spec, top_spec, bot_spec,
                  main_spec, top_spec, bot_spec,
                  main_spec],
        out_specs=main_spec,
        out_shape=jax.ShapeDtypeStruct((H, WL), jnp.float32),
        compiler_params=pltpu.CompilerParams(
            dimension_semantics=("arbitrary",)),
    )(params, img_r, img_r, img_r, ks3, ks3, ks3, nb_r)
    return out.reshape(H, W, C)


def kernel(img, kernel_sizes, readout_noise_base, rgb_QEs, vignet_gain,
           aggregator_gain, dark_currents, noise_gains, STD_reads,
           expsr2dv_gains, expsr2dv_biases, expsr2dv_gamma):
    return _run(img, kernel_sizes, readout_noise_base, rgb_QEs, vignet_gain,
                aggregator_gain, dark_currents, noise_gains, STD_reads,
                expsr2dv_gains, expsr2dv_biases, expsr2dv_gamma)


# speedup vs baseline: 344.0893x; 1.0470x over previous
import functools
import math

import jax
import jax.numpy as jnp
import numpy as np
from jax.experimental import pallas as pl
from jax.experimental.pallas import tpu as pltpu

H = 1080
W = 1920
C = 3
WL = W * C
MAX_HALF_K = 3
BR = 72
HB = BR + 16
NBLK8 = H // 8

_G = np.zeros((8, 4), np.float32)
for _k in range(8):
    if _k <= 1:
        _G[_k, 0] = 1.0
    else:
        _s = _k / 6.0
        _hk = _k // 2
        for _d in range(4):
            if _d <= _hk:
                _G[_k, _d] = math.exp(-_d * _d / (2.0 * _s * _s)) / (
                    math.sqrt(2.0 * math.pi) * _s)


def _gsel(ksf, j):
    w = jnp.full(ksf.shape, _G[7, j], dtype=jnp.float32)
    for k in range(6, -1, -1):
        w = jnp.where(ksf < (k + 0.5), _G[k, j], w)
    return w


def _camera_kernel(scal, params, img_c, img_t, img_b, ks_c, ks_t, ks_b, nb,
                   out_ref):
    i = pl.program_id(0)
    n = pl.num_programs(0)
    vg = scal[0]
    agg = scal[1]
    invg = scal[2]

    qe = params[0:1]
    xx2 = params[1:2]
    dark = params[2:3]
    ngain = params[3:4]
    std = params[4:5]
    egain = params[5:6]
    ebias = params[6:7]

    top_e = jnp.where(i > 0, img_t[:, :], 0.0)
    bot_e = jnp.where(i < n - 1, img_b[:, :], 0.0)
    eh = jnp.concatenate([top_e, img_c[:, :], bot_e], axis=0)
    ksh = jnp.concatenate([ks_t[:, :], ks_c[:, :], ks_b[:, :]], axis=0)

    def to3(a):
        return a.reshape(-1, 8, WL)

    rows = jax.lax.broadcasted_iota(jnp.int32, (HB, WL), 0)
    gy = rows.astype(jnp.float32) + (i * BR - 8).astype(jnp.float32)
    yy = (gy - (H - 1) / 2.0) * (2.0 / W)
    yy3 = to3(yy)
    den = 1.0 + vg * (yy3 * yy3 + xx2)
    rd = jax.lax.rsqrt(den)
    rd2 = rd * rd
    e3 = to3(eh) * (qe * (rd2 * rd2))

    ksh3 = to3(ksh)
    g3 = [_gsel(ksh3, j) for j in range(4)]
    u3 = [e3 * g3[j] for j in range(4)]
    m = {}
    for j in range(4):
        for k in range(j, 4):
            m[(j, k)] = (u3[j] * g3[k]).reshape(HB, WL)

    def mm(a, b):
        return m[(a, b) if a <= b else (b, a)]

    v = []
    for k2 in range(4):
        acc = mm(0, k2)[8:8 + BR, :]
        for d in range(1, 4):
            acc = acc + mm(d, k2)[8 - d:8 - d + BR, :]
            acc = acc + mm(d, k2)[8 + d:8 + d + BR, :]
        v.append(acc)

    lane = jax.lax.broadcasted_iota(jnp.int32, (BR, WL), 1)
    out = v[0]
    for d in range(1, 4):
        s = 3 * d
        out = out + jnp.where(lane >= s, pltpu.roll(v[d], s, 1), 0.0)
        out = out + jnp.where(lane < WL - s, pltpu.roll(v[d], WL - s, 1), 0.0)

    nb3 = to3(nb[:, :])
    eb = to3(out) * agg
    ebc = jnp.maximum(eb, 1e-6)
    shot = ngain * (ebc * jax.lax.rsqrt(ebc)) * nb3
    val = eb + dark + shot + std * nb3
    lin = jnp.maximum(egain * val + ebias, 0.0)
    dv = jnp.exp(jnp.log(lin + 1e-6) * invg)
    out_ref[:, :] = jnp.clip(dv, 0.0, 1.0).reshape(BR, WL)


@jax.jit
def _run(img, kernel_sizes, readout_noise_base, rgb_QEs, vignet_gain,
         aggregator_gain, dark_currents, noise_gains, STD_reads,
         expsr2dv_gains, expsr2dv_biases, expsr2dv_gamma):
    img_r = img.reshape(H, WL)
    nb_r = readout_noise_base.reshape(H, WL)
    ks3 = jnp.repeat(kernel_sizes.astype(jnp.float32), C, axis=1)

    xx = (np.arange(W, dtype=np.float32) - (W - 1) / 2.0) / (W / 2.0)
    xx2 = jnp.asarray(np.repeat(xx * xx, C))
    def tile_c(p):
        return jnp.tile(p.astype(jnp.float32), W)
    params = jnp.stack([
        tile_c(rgb_QEs),
        xx2,
        tile_c(dark_currents),
        tile_c(noise_gains),
        tile_c(STD_reads),
        tile_c(expsr2dv_gains),
        tile_c(expsr2dv_biases),
    ])
    params = jnp.broadcast_to(params[:, None, :], (7, 8, WL))
    scal = jnp.stack([
        jnp.float32(vignet_gain),
        jnp.float32(aggregator_gain),
        1.0 / jnp.float32(expsr2dv_gamma),
    ])

    n = H // BR
    mblk = BR // 8
    main_spec = pl.BlockSpec((BR, WL), lambda i: (i, 0))
    top_spec = pl.BlockSpec((8, WL), lambda i: (jnp.maximum(i * mblk - 1, 0), 0))
    bot_spec = pl.BlockSpec(
        (8, WL), lambda i: (jnp.minimum((i + 1) * mblk, NBLK8 - 1), 0))
    param_spec = pl.BlockSpec((7, 8, WL), lambda i: (0, 0, 0))
    scal_spec = pl.BlockSpec(memory_space=pltpu.SMEM)

    out = pl.pallas_call(
        _camera_kernel,
        grid=(n,),
        in_specs=[scal_spec, param_spec,
                  main_spec, top_spec, bot_spec,
                  main_spec, top_spec, bot_spec,
                  main_spec],
        out_specs=main_spec,
        out_shape=jax.ShapeDtypeStruct((H, WL), jnp.float32),
        compiler_params=pltpu.CompilerParams(
            dimension_semantics=("arbitrary",)),
    )(scal, params, img_r, img_r, img_r, ks3, ks3, ks3, nb_r)
    return out.reshape(H, W, C)


def kernel(img, kernel_sizes, readout_noise_base, rgb_QEs, vignet_gain,
           aggregator_gain, dark_currents, noise_gains, STD_reads,
           expsr2dv_gains, expsr2dv_biases, expsr2dv_gamma):
    return _run(img, kernel_sizes, readout_noise_base, rgb_QEs, vignet_gain,
                aggregator_gain, dark_currents, noise_gains, STD_reads,
                expsr2dv_gains, expsr2dv_biases, expsr2dv_gamma)
